# baseline (speedup 1.0000x reference)
import jax
import jax.numpy as jnp
from jax.experimental import pallas as pl

HEADS = 4


def _gat_conv(x, src, dst, W, a_src, a_dst, bias, out_ch):
    n = x.shape[0]
    h = (x @ W).reshape(n, HEADS, out_ch)
    alpha_src = jnp.sum(h * a_src[None, :, :], axis=-1)
    alpha_dst = jnp.sum(h * a_dst[None, :, :], axis=-1)
    e = alpha_src[src] + alpha_dst[dst]
    e = jax.nn.leaky_relu(e, negative_slope=0.2)
    emax = jax.ops.segment_max(e, dst, num_segments=n)
    ex = jnp.exp(e - emax[dst])
    denom = jax.ops.segment_sum(ex, dst, num_segments=n)
    alpha = ex / (denom[dst] + 1e-16)
    msg = h[src] * alpha[:, :, None]
    out = jax.ops.segment_sum(msg, dst, num_segments=n)
    return out.mean(axis=1) + bias


def _bias_add_kernel(x_ref, o_ref):
    o_ref[...] = x_ref[...]


def kernel(x, edge_index, edge_attr, W1, a1_src, a1_dst, b1, W2, a2_src, a2_dst, b2, W3, a3_src, a3_dst, b3):
    n = x.shape[0]
    src = edge_index[0]
    dst = edge_index[1]
    loop = jnp.arange(n, dtype=src.dtype)
    src = jnp.concatenate([src, loop])
    dst = jnp.concatenate([dst, loop])
    h = _gat_conv(x, src, dst, W1, a1_src, a1_dst, b1, 128)
    h = jax.nn.relu(h)
    h = _gat_conv(h, src, dst, W2, a2_src, a2_dst, b2, 128)
    h = jax.nn.relu(h)
    out = _gat_conv(h, src, dst, W3, a3_src, a3_dst, b3, 1)
    out = pl.pallas_call(
        _bias_add_kernel,
        out_shape=jax.ShapeDtypeStruct(out.shape, out.dtype),
    )(out)
    return out


# speedup vs baseline: 16.2615x; 16.2615x over previous
import functools

import jax
import jax.numpy as jnp
from jax import lax
from jax.experimental import pallas as pl
from jax.experimental.pallas import tpu as pltpu
from jax.experimental.pallas import tpu_sc as plsc

HH = 4
NC = 2
NS = 16
NW = NC * NS
LL = 16
WIN = 16
NEG = -1e30

f32 = jnp.float32
i32 = jnp.int32


def _rup(a, b):
    return (a + b - 1) // b * b


def _tc_first_body(x_ref, w_ref, a_ref, h_ref, aa_ref, gp_ref):
    h = jnp.dot(x_ref[...], w_ref[...], preferred_element_type=f32,
                precision=jax.lax.Precision.HIGHEST)
    h_ref[...] = h
    aa = jnp.dot(h, a_ref[...], preferred_element_type=f32,
                 precision=jax.lax.Precision.HIGHEST)
    aa_ref[...] = aa
    gp_ref[0, 0, :] = jnp.max(aa, axis=0)


def _tc_mid_body(p0_ref, p1_ref, b_ref, w_ref, a_ref, h_ref, aa_ref, gp_ref):
    z = jnp.maximum(p0_ref[...] + p1_ref[...] + b_ref[...], 0.0)
    h = jnp.dot(z, w_ref[...], preferred_element_type=f32,
                precision=jax.lax.Precision.HIGHEST)
    h_ref[...] = h
    aa = jnp.dot(h, a_ref[...], preferred_element_type=f32,
                 precision=jax.lax.Precision.HIGHEST)
    aa_ref[...] = aa
    gp_ref[0, 0, :] = jnp.max(aa, axis=0)


@functools.lru_cache(maxsize=None)
def _make_tc_first(npad, tr, kdim, cout):
    grid = npad // tr
    return pl.pallas_call(
        _tc_first_body,
        grid=(grid,),
        in_specs=[
            pl.BlockSpec((tr, kdim), lambda i: (i, 0)),
            pl.BlockSpec((kdim, cout), lambda i: (0, 0)),
            pl.BlockSpec((cout, 8), lambda i: (0, 0)),
        ],
        out_specs=[
            pl.BlockSpec((tr, cout), lambda i: (i, 0)),
            pl.BlockSpec((tr, 8), lambda i: (i, 0)),
            pl.BlockSpec((1, 1, 8), lambda i: (i, 0, 0)),
        ],
        out_shape=[
            jax.ShapeDtypeStruct((npad, cout), f32),
            jax.ShapeDtypeStruct((npad, 8), f32),
            jax.ShapeDtypeStruct((grid, 1, 8), f32),
        ],
    )


@functools.lru_cache(maxsize=None)
def _make_tc_mid(npad, tr, kdim, cout):
    grid = npad // tr
    return pl.pallas_call(
        _tc_mid_body,
        grid=(grid,),
        in_specs=[
            pl.BlockSpec((tr, kdim), lambda i: (i, 0)),
            pl.BlockSpec((tr, kdim), lambda i: (i, 0)),
            pl.BlockSpec((1, kdim), lambda i: (0, 0)),
            pl.BlockSpec((kdim, cout), lambda i: (0, 0)),
            pl.BlockSpec((cout, 8), lambda i: (0, 0)),
        ],
        out_specs=[
            pl.BlockSpec((tr, cout), lambda i: (i, 0)),
            pl.BlockSpec((tr, 8), lambda i: (i, 0)),
            pl.BlockSpec((1, 1, 8), lambda i: (i, 0, 0)),
        ],
        out_shape=[
            jax.ShapeDtypeStruct((npad, cout), f32),
            jax.ShapeDtypeStruct((npad, 8), f32),
            jax.ShapeDtypeStruct((grid, 1, 8), f32),
        ],
    )


@functools.lru_cache(maxsize=None)
def _make_sc_att(npad, ce):
    nwin = ce // WIN
    rows = npad // NS
    mesh = plsc.VectorSubcoreMesh(
        core_axis_name="c", subcore_axis_name="s", num_cores=NC, num_subcores=NS
    )

    def body(src_h, dst_h, asf_h, adf_h, g_h,
             s_out_h, t_out_h,
             asf_v, adf_v, g_v, src_v, dst_v, tb_v, tc_v, dw_v, zb_v, s_sh):
        cid = lax.axis_index("c")
        sid = lax.axis_index("s")
        wid = sid * NC + cid
        base = wid * ce
        pltpu.sync_copy(asf_h, asf_v)
        pltpu.sync_copy(adf_h, adf_v)
        pltpu.sync_copy(g_h, g_v)
        pltpu.sync_copy(src_h.at[pl.ds(base, ce)], src_v)
        pltpu.sync_copy(dst_h.at[pl.ds(base, ce)], dst_v)

        iota = lax.iota(i32, LL)
        zeros = jnp.zeros((LL,), f32)

        def zrow(r, _):
            plsc.store_scatter(zb_v, [jnp.full((LL,), 0, i32) + r, iota], zeros)
            return 0
        lax.fori_loop(0, rows, zrow, 0)
        for r in range(WIN):
            plsc.store_scatter(tb_v, [jnp.full((LL,), r, i32), iota], zeros)
        pltpu.sync_copy(zb_v, s_sh.at[pl.ds(sid * rows, rows)])
        plsc.subcore_barrier()

        def win(w, _):
            eb = w * WIN
            srcv = src_v[pl.ds(eb, LL)]
            dstv = dst_v[pl.ds(eb, LL)]
            s4 = srcv * HH
            d4 = dstv * HH
            for h in range(HH):
                asv = plsc.load_gather(asf_v, [s4 + h])
                adv = plsc.load_gather(adf_v, [d4 + h])
                gh = plsc.load_gather(g_v, [jnp.full((LL,), h + 1, i32)])
                e = asv + adv
                e = jnp.maximum(e, 0.2 * e)
                bb = gh + adv
                bb = jnp.maximum(bb, 0.2 * bb)
                t = jnp.exp(e - bb)
                plsc.store_scatter(tb_v, [iota, jnp.full((LL,), h, i32)], t)
                plsc.store_scatter(tc_v, [iota, jnp.full((LL,), h, i32)], t)
            dw_v[...] = dstv
            pltpu.sync_copy(tb_v, s_sh.at[dw_v], add=True)
            pltpu.sync_copy(tc_v, t_out_h.at[pl.ds(base + eb, WIN)])
            return 0
        lax.fori_loop(0, nwin, win, 0)
        plsc.subcore_barrier()

        pltpu.sync_copy(s_sh.at[pl.ds(sid * rows, rows)], zb_v)
        pltpu.sync_copy(zb_v, s_out_h.at[cid, pl.ds(sid * rows, rows)])

    return pl.kernel(
        body,
        out_type=[
            jax.ShapeDtypeStruct((NC, npad, LL), f32),
            jax.ShapeDtypeStruct((ce * NW, HH), f32),
        ],
        mesh=mesh,
        compiler_params=pltpu.CompilerParams(needs_layout_passes=False, use_tc_tiling_on_sc=False),
        scratch_types=[
            pltpu.VMEM((HH * npad,), f32),
            pltpu.VMEM((HH * npad,), f32),
            pltpu.VMEM((LL,), f32),
            pltpu.VMEM((ce,), i32),
            pltpu.VMEM((ce,), i32),
            pltpu.VMEM((WIN, LL), f32),
            pltpu.VMEM((WIN, HH), f32),
            pltpu.VMEM((LL,), i32),
            pltpu.VMEM((npad // NS, LL), f32),
            pltpu.VMEM_SHARED((npad, LL), f32),
        ],
    )


@functools.lru_cache(maxsize=None)
def _make_sc_msg(npad, ce, cdim):
    nwin = ce // WIN
    rows = npad // NS
    zrows = 16
    mesh = plsc.VectorSubcoreMesh(
        core_axis_name="c", subcore_axis_name="s", num_cores=NC, num_subcores=NS
    )
    ncv = cdim // LL

    def body(src_h, dst_h, t_h, invf_h, hf_h,
             o_out_h,
             src_v, dst_v, rows_v, tw_v, m_v, al_v, ib_v, iv_v, dw_v, zb_v,
             o_sh):
        cid = lax.axis_index("c")
        sid = lax.axis_index("s")
        wid = sid * NC + cid
        base = wid * ce
        pltpu.sync_copy(src_h.at[pl.ds(base, ce)], src_v)
        pltpu.sync_copy(dst_h.at[pl.ds(base, ce)], dst_v)

        iota = lax.iota(i32, LL)
        zeros = jnp.zeros((LL,), f32)

        def zrow(r, _):
            for c in range(ncv):
                plsc.store_scatter(
                    zb_v, [jnp.full((LL,), 0, i32) + r, c * LL + iota], zeros)
            return 0
        lax.fori_loop(0, zrows, zrow, 0)
        for r8 in range(rows // zrows):
            pltpu.sync_copy(
                zb_v, o_sh.at[pl.ds(sid * rows + r8 * zrows, zrows)])
        plsc.subcore_barrier()

        def win(w, _):
            eb = w * WIN
            dstv = dst_v[pl.ds(eb, LL)]
            d4 = dstv * HH
            for h in range(HH):
                ib_v[pl.ds(h * LL, LL)] = d4 + h
            pltpu.sync_copy(invf_h.at[ib_v], iv_v)
            pltpu.sync_copy(hf_h.at[src_v.at[pl.ds(eb, WIN)]], rows_v)
            pltpu.sync_copy(t_h.at[pl.ds(base + eb, WIN)], tw_v)
            for h in range(HH):
                tv = plsc.load_gather(tw_v, [iota, jnp.full((LL,), h, i32)])
                al_v[pl.ds((h + 1) * LL, LL)] = tv * iv_v[pl.ds(h * LL, LL)]
            for k in range(WIN):
                bc = [
                    plsc.load_gather(
                        al_v, [jnp.full((LL,), (h + 1) * LL + k, i32)])
                    for h in range(HH)
                ]
                for c in range(ncv):
                    acc = bc[0] * rows_v[k, pl.ds(c * LL, LL)]
                    for h in range(1, HH):
                        acc = acc + bc[h] * rows_v[k, pl.ds(h * cdim + c * LL, LL)]
                    m_v[k, pl.ds(c * LL, LL)] = acc
            dw_v[...] = dstv
            pltpu.sync_copy(m_v, o_sh.at[dw_v], add=True)
            return 0
        lax.fori_loop(0, nwin, win, 0)
        plsc.subcore_barrier()

        for r8 in range(rows // zrows):
            pltpu.sync_copy(
                o_sh.at[pl.ds(sid * rows + r8 * zrows, zrows)], zb_v)
            pltpu.sync_copy(
                zb_v, o_out_h.at[cid, pl.ds(sid * rows + r8 * zrows, zrows)])

    return pl.kernel(
        body,
        out_type=jax.ShapeDtypeStruct((NC, npad, cdim), f32),
        mesh=mesh,
        compiler_params=pltpu.CompilerParams(needs_layout_passes=False, use_tc_tiling_on_sc=False),
        scratch_types=[
            pltpu.VMEM((ce,), i32),
            pltpu.VMEM((ce,), i32),
            pltpu.VMEM((WIN, HH * cdim), f32),
            pltpu.VMEM((WIN, HH), f32),
            pltpu.VMEM((WIN, cdim), f32),
            pltpu.VMEM(((HH + 1) * LL,), f32),
            pltpu.VMEM((HH * LL,), i32),
            pltpu.VMEM((HH * LL,), f32),
            pltpu.VMEM((LL,), i32),
            pltpu.VMEM((zrows, cdim), f32),
            pltpu.VMEM_SHARED((npad, cdim), f32),
        ],
    )


@functools.lru_cache(maxsize=None)
def _make_sc_fin(npad, ce):
    nwin = ce // WIN
    rows = npad // NS
    mesh = plsc.VectorSubcoreMesh(
        core_axis_name="c", subcore_axis_name="s", num_cores=NC, num_subcores=NS
    )

    def body(src_h, dst_h, t_h, invf_h, h3f_h,
             o_out_h,
             inv_v, h3_v, src_v, dst_v, tw_v, tb_v, dw_v, zb_v, o_sh):
        cid = lax.axis_index("c")
        sid = lax.axis_index("s")
        wid = sid * NC + cid
        base = wid * ce
        pltpu.sync_copy(invf_h, inv_v)
        pltpu.sync_copy(h3f_h, h3_v)
        pltpu.sync_copy(src_h.at[pl.ds(base, ce)], src_v)
        pltpu.sync_copy(dst_h.at[pl.ds(base, ce)], dst_v)

        iota = lax.iota(i32, LL)
        zeros = jnp.zeros((LL,), f32)

        def zrow(r, _):
            plsc.store_scatter(zb_v, [jnp.full((LL,), 0, i32) + r, iota], zeros)
            return 0
        lax.fori_loop(0, rows, zrow, 0)
        for r in range(WIN):
            plsc.store_scatter(tb_v, [jnp.full((LL,), r, i32), iota], zeros)
        pltpu.sync_copy(zb_v, o_sh.at[pl.ds(sid * rows, rows)])
        plsc.subcore_barrier()

        def win(w, _):
            eb = w * WIN
            pltpu.sync_copy(t_h.at[pl.ds(base + eb, WIN)], tw_v)
            srcv = src_v[pl.ds(eb, LL)]
            dstv = dst_v[pl.ds(eb, LL)]
            s4 = srcv * HH
            d4 = dstv * HH
            acc = jnp.zeros((LL,), f32)
            for h in range(HH):
                tv = plsc.load_gather(tw_v, [iota, jnp.full((LL,), h, i32)])
                iv = plsc.load_gather(inv_v, [d4 + h])
                hv = plsc.load_gather(h3_v, [s4 + h])
                acc = acc + tv * iv * hv
            plsc.store_scatter(tb_v, [iota, jnp.full((LL,), 0, i32)], acc)
            dw_v[...] = dstv
            pltpu.sync_copy(tb_v, o_sh.at[dw_v], add=True)
            return 0
        lax.fori_loop(0, nwin, win, 0)
        plsc.subcore_barrier()

        pltpu.sync_copy(o_sh.at[pl.ds(sid * rows, rows)], zb_v)
        pltpu.sync_copy(zb_v, o_out_h.at[cid, pl.ds(sid * rows, rows)])

    return pl.kernel(
        body,
        out_type=jax.ShapeDtypeStruct((NC, npad, LL), f32),
        mesh=mesh,
        compiler_params=pltpu.CompilerParams(needs_layout_passes=False, use_tc_tiling_on_sc=False),
        scratch_types=[
            pltpu.VMEM((HH * npad,), f32),
            pltpu.VMEM((HH * npad,), f32),
            pltpu.VMEM((ce,), i32),
            pltpu.VMEM((ce,), i32),
            pltpu.VMEM((WIN, HH), f32),
            pltpu.VMEM((WIN, LL), f32),
            pltpu.VMEM((LL,), i32),
            pltpu.VMEM((npad // NS, LL), f32),
            pltpu.VMEM_SHARED((npad, LL), f32),
        ],
    )


def _mix_matrix(a_src, a_dst, cout):
    eye = jnp.eye(HH, dtype=f32)
    asrc = (eye[:, None, :] * a_src[:, :, None]).reshape(HH * cout, HH)
    adst = (eye[:, None, :] * a_dst[:, :, None]).reshape(HH * cout, HH)
    return jnp.concatenate([asrc, adst], axis=1)


def _att_tables(aa, gp, n, npad):
    asf = jnp.concatenate(
        [aa[:n, 0:HH], jnp.full((npad - n, HH), NEG, f32)]).reshape(-1)
    adf = aa[:, HH:2 * HH].reshape(-1)
    g = jnp.max(gp.reshape(-1, 8), axis=0)[:HH]
    g16 = jnp.concatenate([jnp.zeros((1,), f32), g,
                           jnp.zeros((LL - HH - 1,), f32)])
    return asf, adf, g16


def _inv_table(s_out):
    s = s_out[0] + s_out[1]
    return (0.25 / (s[:, :HH] + 1e-16)).reshape(-1)


def kernel(x, edge_index, edge_attr, W1, a1_src, a1_dst, b1,
           W2, a2_src, a2_dst, b2, W3, a3_src, a3_dst, b3):
    n = x.shape[0]
    e = edge_index.shape[1]
    npad = _rup(n + 1, 256)
    etot = e + n
    ce = _rup(-(-etot // NW), WIN)
    ep = ce * NW

    nwing = ep // WIN
    loop = jnp.arange(n, dtype=i32)
    src_all = jnp.concatenate([edge_index[0], loop,
                               jnp.full((ep - etot,), n, i32)])
    dst_all = jnp.concatenate([edge_index[1], loop,
                               jnp.full((ep - etot,), n, i32)])
    deal = jnp.argsort(dst_all).reshape(WIN, nwing).T.reshape(-1)
    srcp = src_all[deal]
    dstp = dst_all[deal]

    tr = 256
    tck1 = _make_tc_first(npad, tr, 32, HH * 128)
    tck2 = _make_tc_mid(npad, tr, 128, HH * 128)
    tck3 = _make_tc_mid(npad, tr, 128, 8)
    sca = _make_sc_att(npad, ce)
    scb = _make_sc_msg(npad, ce, 128)
    scc = _make_sc_fin(npad, ce)

    xp = jnp.zeros((npad, 32), f32).at[:n, :x.shape[1]].set(x)
    w1p = jnp.zeros((32, HH * 128), f32).at[:W1.shape[0]].set(W1)
    h1, aa1, gp1 = tck1(xp, w1p, _mix_matrix(a1_src, a1_dst, 128))
    asf1, adf1, g1 = _att_tables(aa1, gp1, n, npad)
    s1, t1 = sca(srcp, dstp, asf1, adf1, g1)
    o1 = scb(srcp, dstp, t1, _inv_table(s1), h1)

    h2, aa2, gp2 = tck2(o1[0], o1[1], b1[None, :], W2,
                        _mix_matrix(a2_src, a2_dst, 128))
    asf2, adf2, g2 = _att_tables(aa2, gp2, n, npad)
    s2, t2 = sca(srcp, dstp, asf2, adf2, g2)
    o2 = scb(srcp, dstp, t2, _inv_table(s2), h2)

    w3p = jnp.zeros((128, 8), f32).at[:, :HH].set(W3)
    a3s = jnp.zeros((HH, 8), f32).at[:, :HH].set(jnp.diag(a3_src[:, 0]))
    a3d = jnp.zeros((HH, 8), f32).at[:, HH:].set(jnp.diag(a3_dst[:, 0]))
    mix3 = jnp.concatenate([a3s + a3d, jnp.zeros((4, 8), f32)], axis=0)
    h3, aa3, gp3 = tck3(o2[0], o2[1], b2[None, :], w3p, mix3)
    asf3, adf3, g3 = _att_tables(aa3, gp3, n, npad)
    s3, t3 = sca(srcp, dstp, asf3, adf3, g3)
    h3f = h3[:, :HH].reshape(-1)
    o3 = scc(srcp, dstp, t3, _inv_table(s3), h3f)

    return (o3[0] + o3[1])[:n, 0:1] + b3[None, :]
